# Initial kernel scaffold; baseline (speedup 1.0000x reference)
import jax
import jax.numpy as jnp
from jax.experimental import pallas as pl


def kernel(seq, w_write, b_write, w_dem, b_dem, Wq, bq, Wout, bout):
    raise NotImplementedError("write your pallas kernel here")


# speedup vs baseline: 52.5791x; 52.5791x over previous
import jax
import jax.numpy as jnp
from jax import lax
from jax.experimental import pallas as pl
from jax.experimental.pallas import tpu as pltpu

F32 = jnp.float32
I32 = jnp.int32
_HIGH = lax.Precision.HIGHEST


def _dot(a, b, dims, precision=_HIGH):
    return lax.dot_general(a, b, (dims, ((), ())), precision=precision,
                           preferred_element_type=F32)


def _body(seq_ref, ww_ref, wd_ref, bw_ref, bd_ref, wq_ref, bq_ref,
          wo_ref, bo_ref, logits_ref, smask_ref, sT_ref, dT_ref):
    B, T, H = seq_ref.shape
    FAST = 64
    SLOW = 256
    STEPS = T - 3
    N = FAST + SLOW

    bw = bw_ref[0, 0]
    bd = bd_ref[0, 0]

    ww16 = jnp.broadcast_to(ww_ref[...], (2 * B, H))
    wd16 = jnp.broadcast_to(wd_ref[...], (2 * B, H))
    iota2b = lax.broadcasted_iota(I32, (2 * B, H), 0)
    sd = jnp.zeros((T, 2 * B), F32)
    for b in range(B):
        rhs = (jnp.where(iota2b == b, ww16, 0.0)
               + jnp.where(iota2b == B + b, wd16, 0.0))
        sd = sd + _dot(seq_ref[b], rhs, ((1,), (1,)))
    sT_ref[...] = 1.0 / (1.0 + jnp.exp(-(sd[:, 0:B] + bw)))
    dT_ref[...] = sd[:, B:2 * B] + bd

    eyeB = (lax.broadcasted_iota(I32, (B, B), 0)
            == lax.broadcasted_iota(I32, (B, B), 1))
    zf = jnp.zeros((B, B), F32)

    def lane2sub(x):
        return jnp.sum(jnp.where(eyeB, jnp.broadcast_to(x, (B, B)), zf),
                       axis=1, keepdims=True)

    iota_f = lax.broadcasted_iota(I32, (B, FAST), 1)
    iota_s = lax.broadcasted_iota(I32, (B, SLOW), 1)
    BIG = jnp.float32(3.0e38)

    def step(t, carry):
        fd, ftok, stok, swt, nf, ns = carry
        s_t = lane2sub(sT_ref[pl.ds(t, 1), :])
        d_t = lane2sub(dT_ref[pl.ds(t, 1), :])
        write = ~(s_t < 0.4)
        full_f = nf >= FAST
        dmin = jnp.min(fd, axis=1, keepdims=True)
        ds_ = jnp.min(jnp.where(fd == dmin, iota_f, FAST),
                      axis=1, keepdims=True)
        fslot = jnp.where(full_f, ds_, nf)
        demoted = jnp.min(jnp.where(iota_f == ds_, ftok, T),
                          axis=1, keepdims=True)
        full_s = ns >= SLOW
        wmin = jnp.min(swt, axis=1, keepdims=True)
        ss_e = jnp.min(jnp.where(swt == wmin, iota_s, SLOW),
                       axis=1, keepdims=True)
        ss = jnp.where(full_s, ss_e, ns)
        do_slow = write & full_f
        upd_s = do_slow & (iota_s == ss)
        stok = jnp.where(upd_s, demoted, stok)
        swt = jnp.where(upd_s, t, swt)
        ns = ns + (do_slow & ~full_s).astype(I32)
        upd_f = write & (iota_f == fslot)
        fd = jnp.where(upd_f, d_t, fd)
        ftok = jnp.where(upd_f, t, ftok)
        nf = nf + (write & ~full_f).astype(I32)
        return fd, ftok, stok, swt, nf, ns

    carry = (jnp.full((B, FAST), BIG, F32),
             jnp.zeros((B, FAST), I32),
             jnp.zeros((B, SLOW), I32),
             jnp.zeros((B, SLOW), I32),
             jnp.zeros((B, 1), I32),
             jnp.zeros((B, 1), I32))
    _, ftok, stok, _, nf, ns = lax.fori_loop(0, STEPS, step, carry)

    q = _dot(seq_ref[:, T - 1, :], wq_ref[...], ((1,), (1,))) + bq_ref[...]

    iota_bh = lax.broadcasted_iota(I32, (B, H), 0)
    pT = jnp.zeros((T, B), F32)
    for b in range(B):
        qb = jnp.where(iota_bh == b, jnp.broadcast_to(q[b:b + 1, :], (B, H)),
                       0.0)
        pT = pT + _dot(seq_ref[b], qb, ((1,), (1,)))

    tok = jnp.concatenate([ftok, stok], axis=1)
    iota_n = lax.broadcasted_iota(I32, (B, N), 1)
    used = ((iota_n < FAST) & (iota_n < nf)) | ((iota_n >= FAST)
                                               & (iota_n - FAST < ns))

    sub_n = lax.broadcasted_iota(I32, (B, N), 0)
    iota_tn = lax.broadcasted_iota(I32, (T, N), 0)
    scores = jnp.zeros((B, N), F32)
    for b in range(B):
        ohT_b = (iota_tn == jnp.broadcast_to(tok[b:b + 1, :], (T, N))
                 ).astype(F32)
        prod = _dot(pT, ohT_b, ((0,), (0,)))
        scores = scores + jnp.where(sub_n == b, prod, 0.0)

    scores = jnp.where(used, scores, -1.0e9)
    m = jnp.max(scores, axis=1, keepdims=True)
    e = jnp.exp(scores - m)
    attn = e / jnp.sum(e, axis=1, keepdims=True)
    attn = jnp.where(used, attn, 0.0)

    lane_tb = lax.broadcasted_iota(I32, (T, B), 1)
    W = jnp.zeros((T, B), F32)
    for b in range(B):
        ohT_b = (iota_tn == jnp.broadcast_to(tok[b:b + 1, :], (T, N))
                 ).astype(F32)
        prod = _dot(ohT_b, attn, ((1,), (1,)))
        W = W + jnp.where(lane_tb == b, prod, 0.0)

    sub_h = lax.broadcasted_iota(I32, (B, H), 0)
    ctx = jnp.zeros((B, H), F32)
    for b in range(B):
        prod = _dot(W, seq_ref[b], ((0,), (0,)))
        ctx = ctx + jnp.where(sub_h == b, prod, 0.0)

    logits_ref[...] = _dot(ctx, wo_ref[...], ((1,), (1,))) + bo_ref[...]
    smask_ref[...] = (iota_s < ns).astype(F32)


@jax.jit
def kernel(seq, w_write, b_write, w_dem, b_dem, Wq, bq, Wout, bout):
    B, T, H = seq.shape
    SLOW = 256
    VOCAB = Wout.shape[0]
    out_shape = [jax.ShapeDtypeStruct((B, VOCAB), F32),
                 jax.ShapeDtypeStruct((B, SLOW), F32)]
    vspec = pl.BlockSpec(memory_space=pltpu.VMEM)
    sspec = pl.BlockSpec(memory_space=pltpu.SMEM)
    logits, slow_mask = pl.pallas_call(
        _body,
        out_shape=out_shape,
        in_specs=[vspec, vspec, vspec, sspec, sspec, vspec, vspec, vspec,
                  vspec],
        scratch_shapes=[pltpu.VMEM((T, B), F32), pltpu.VMEM((T, B), F32)],
        compiler_params=pltpu.CompilerParams(
            vmem_limit_bytes=100 * 1024 * 1024),
    )(seq, w_write, w_dem,
      b_write.reshape(1, 1), b_dem.reshape(1, 1),
      Wq, bq.reshape(1, H), Wout, bout.reshape(1, VOCAB))
    return logits, slow_mask


# speedup vs baseline: 164.9190x; 3.1366x over previous
import functools

import jax
import jax.numpy as jnp
from jax import lax
from jax.experimental import pallas as pl
from jax.experimental.pallas import tpu as pltpu
from jax.experimental.pallas import tpu_sc as plsc

F32 = jnp.float32
I32 = jnp.int32
_HIGH = lax.Precision.HIGHEST

FAST = 64
SLOW = 256
MAXI = 0x7FFFFFFF
NCORE = 2
NSUB = 16


def _dot(a, b, dims, precision=_HIGH):
    return lax.dot_general(a, b, (dims, ((), ())), precision=precision,
                           preferred_element_type=F32)


def _proj_body(seq_ref, ww_ref, wd_ref, bw_ref, bd_ref, wq_ref, bq_ref,
               encT_ref, pT_ref):
    B, T, H = seq_ref.shape
    bw = bw_ref[0, 0]
    bd = bd_ref[0, 0]

    ww16 = jnp.broadcast_to(ww_ref[...], (2 * B, H))
    wd16 = jnp.broadcast_to(wd_ref[...], (2 * B, H))
    iota2b = lax.broadcasted_iota(I32, (2 * B, H), 0)
    sd = jnp.zeros((T, 2 * B), F32)
    for b in range(B):
        rhs = (jnp.where(iota2b == b, ww16, 0.0)
               + jnp.where(iota2b == B + b, wd16, 0.0))
        sd = sd + _dot(seq_ref[b], rhs, ((1,), (1,)))
    s = 1.0 / (1.0 + jnp.exp(-(sd[:, 0:B] + bw)))
    d = sd[:, B:2 * B] + bd

    bits = lax.bitcast_convert_type(d, I32)
    key = jnp.where(bits < 0, (~bits) ^ jnp.int32(-2147483648), bits)
    write = ~(s < 0.4)
    encT_ref[...] = jnp.where(write, key, jnp.int32(MAXI))

    q = _dot(seq_ref[:, T - 1, :], wq_ref[...], ((1,), (1,)),
             precision=None) + bq_ref[...]
    iota_bh = lax.broadcasted_iota(I32, (B, H), 0)
    pT = jnp.zeros((T, B), F32)
    for b in range(B):
        qb = jnp.where(iota_bh == b, jnp.broadcast_to(q[b:b + 1, :], (B, H)),
                       0.0)
        pT = pT + _dot(seq_ref[b], qb, ((1,), (1,)), precision=None)
    pT_ref[...] = pT


def _sc_body(enc_hbm, ftok_hbm, stok_hbm, cnt_hbm,
             enc_v, fd_v, ftok_v, stok_v, cnt_v, sem):
    b = lax.axis_index("s") * NCORE + lax.axis_index("c")
    B, T = enc_hbm.shape
    STEPS = T - 3

    @pl.when(b < B)
    def _():
        pltpu.sync_copy(enc_hbm.at[b], enc_v.at[pl.ds(0, T)])
        for j in range(FAST // 16):
            fd_v[pl.ds(16 * j, 16)] = jnp.full((16,), MAXI, I32)
        iota16 = lax.broadcasted_iota(I32, (16,), 0)
        lane0 = iota16 == 0

        def vmin_all(x):
            for k in (8, 4, 2, 1):
                sh = x.at[iota16 ^ k].get(mode="promise_in_bounds")
                x = jnp.minimum(x, sh)
            return x[0]

        def put1(ref, idx, val):
            plsc.store_scatter(ref, [jnp.full((16,), idx, I32)],
                               jnp.full((16,), val, I32), mask=lane0)

        def step(t, carry):
            nf, ns = carry
            e = enc_v[pl.ds(t, 16)][0]

            def no_write(nf, ns):
                return nf, ns

            def do_write(nf, ns):
                def not_full(nf, ns):
                    put1(fd_v, nf, e)
                    put1(ftok_v, nf, t)
                    return nf + 1, ns

                def full(nf, ns):
                    vs = [fd_v[pl.ds(16 * j, 16)] for j in range(4)]
                    m = jnp.minimum(jnp.minimum(vs[0], vs[1]),
                                    jnp.minimum(vs[2], vs[3]))
                    gmin = vmin_all(m)
                    em_vec = jnp.full((16,), MAXI, I32)
                    for j in range(4):
                        tk = ftok_v[pl.ds(16 * j, 16)]
                        enc2 = jnp.where(vs[j] == gmin,
                                         (iota16 + 16 * j) * 1024 + tk,
                                         jnp.int32(MAXI))
                        em_vec = jnp.minimum(em_vec, enc2)
                    em = vmin_all(em_vec)
                    ds_ = em >> 10
                    demoted = em & 1023
                    ss = ns & (SLOW - 1)
                    put1(stok_v, ss, demoted)
                    put1(fd_v, ds_, e)
                    put1(ftok_v, ds_, t)
                    return nf, ns + 1

                return lax.cond(nf >= FAST, full, not_full, nf, ns)

            return lax.cond(e != jnp.int32(MAXI), do_write, no_write, nf, ns)

        nf, ns = lax.fori_loop(0, STEPS, step,
                               (jnp.int32(0), jnp.int32(0)))
        cnt_vec = jnp.where(lane0, jnp.full((16,), nf, I32),
                            jnp.where(iota16 == 1,
                                      jnp.full((16,), jnp.minimum(ns, SLOW),
                                               I32),
                                      jnp.zeros((16,), I32)))
        cnt_v[pl.ds(0, 16)] = cnt_vec
        pltpu.sync_copy(ftok_v, ftok_hbm.at[b])
        pltpu.sync_copy(stok_v, stok_hbm.at[b])
        pltpu.sync_copy(cnt_v, cnt_hbm.at[b])


def _attn_body(seq_ref, pT_ref, ftok_ref, stok_ref, cnt_ref, wo_ref, bo_ref,
               logits_ref, smask_ref):
    B, T, H = seq_ref.shape
    N = FAST + SLOW

    nf = cnt_ref[:, 0:1]
    ns = cnt_ref[:, 1:2]
    tok = jnp.concatenate([ftok_ref[...], stok_ref[...]], axis=1)
    iota_n = lax.broadcasted_iota(I32, (B, N), 1)
    used = ((iota_n < FAST) & (iota_n < nf)) | ((iota_n >= FAST)
                                               & (iota_n - FAST < ns))

    pT = pT_ref[...]
    sub_n = lax.broadcasted_iota(I32, (B, N), 0)
    iota_tn = lax.broadcasted_iota(I32, (T, N), 0)
    scores = jnp.zeros((B, N), F32)
    for b in range(B):
        ohT_b = (iota_tn == jnp.broadcast_to(tok[b:b + 1, :], (T, N))
                 ).astype(F32)
        prod = _dot(pT, ohT_b, ((0,), (0,)))
        scores = scores + jnp.where(sub_n == b, prod, 0.0)

    scores = jnp.where(used, scores, -1.0e9)
    m = jnp.max(scores, axis=1, keepdims=True)
    e = jnp.exp(scores - m)
    attn = e / jnp.sum(e, axis=1, keepdims=True)
    attn = jnp.where(used, attn, 0.0)

    lane_tb = lax.broadcasted_iota(I32, (T, B), 1)
    W = jnp.zeros((T, B), F32)
    for b in range(B):
        ohT_b = (iota_tn == jnp.broadcast_to(tok[b:b + 1, :], (T, N))
                 ).astype(F32)
        prod = _dot(ohT_b, attn, ((1,), (1,)))
        W = W + jnp.where(lane_tb == b, prod, 0.0)

    sub_h = lax.broadcasted_iota(I32, (B, H), 0)
    ctx = jnp.zeros((B, H), F32)
    for b in range(B):
        prod = _dot(W, seq_ref[b], ((0,), (0,)))
        ctx = ctx + jnp.where(sub_h == b, prod, 0.0)

    logits_ref[...] = _dot(ctx, wo_ref[...], ((1,), (1,)),
                           precision=None) + bo_ref[...]
    iota_s = lax.broadcasted_iota(I32, (B, SLOW), 1)
    smask_ref[...] = (iota_s < ns).astype(F32)


@jax.jit
def kernel(seq, w_write, b_write, w_dem, b_dem, Wq, bq, Wout, bout):
    B, T, H = seq.shape
    VOCAB = Wout.shape[0]
    vspec = pl.BlockSpec(memory_space=pltpu.VMEM)
    sspec = pl.BlockSpec(memory_space=pltpu.SMEM)

    encT, pT = pl.pallas_call(
        _proj_body,
        out_shape=[jax.ShapeDtypeStruct((T, B), I32),
                   jax.ShapeDtypeStruct((T, B), F32)],
        in_specs=[vspec, vspec, vspec, sspec, sspec, vspec, vspec],
        compiler_params=pltpu.CompilerParams(
            vmem_limit_bytes=100 * 1024 * 1024),
    )(seq, w_write, w_dem, b_write.reshape(1, 1), b_dem.reshape(1, 1),
      Wq, bq.reshape(1, H))

    enc = encT.T

    sc_machine = functools.partial(
        pl.kernel,
        out_type=[jax.ShapeDtypeStruct((B, FAST), I32),
                  jax.ShapeDtypeStruct((B, SLOW), I32),
                  jax.ShapeDtypeStruct((B, 16), I32)],
        mesh=plsc.VectorSubcoreMesh(core_axis_name="c", subcore_axis_name="s"),
        compiler_params=pltpu.CompilerParams(needs_layout_passes=False),
        scratch_types=[pltpu.VMEM((T + 16,), I32),
                       pltpu.VMEM((FAST,), I32),
                       pltpu.VMEM((FAST,), I32),
                       pltpu.VMEM((SLOW,), I32),
                       pltpu.VMEM((16,), I32),
                       pltpu.SemaphoreType.DMA],
    )(_sc_body)
    ftok, stok, cnt = sc_machine(enc)

    logits, smask = pl.pallas_call(
        _attn_body,
        out_shape=[jax.ShapeDtypeStruct((B, VOCAB), F32),
                   jax.ShapeDtypeStruct((B, SLOW), F32)],
        in_specs=[vspec] * 7,
        compiler_params=pltpu.CompilerParams(
            vmem_limit_bytes=100 * 1024 * 1024),
    )(seq, pT, ftok, stok, cnt, Wout, bout.reshape(1, VOCAB))
    return logits, smask
